# baseline (speedup 1.0000x reference)
import functools

import jax
import jax.numpy as jnp
from jax import lax
from jax.experimental import pallas as pl
from jax.experimental.pallas import tpu as pltpu
from jax.experimental.pallas import tpu_sc as plsc

D_OUT = 439
D_EMB = 216
D_SCAL = 223
C = 128
G = C // 16

_FIELDS = (
    (1, 32, 8, 0),
    (2, 40, 8, 0),
    (3, 48, 8, 0),
    (4, 56, 16, 1),
    (5, 72, 16, 2),
    (6, 88, 32, 3),
    (7, 120, 32, 3),
    (8, 152, 32, 3),
    (9, 184, 32, 3),
)


def _make_kernel(rows: int):
    info = plsc.get_sparse_core_info()
    nw = info.num_cores * info.num_subcores
    assert rows % (nw * 2 * C) == 0
    rows_per_w = rows // nw
    n_chunks = rows_per_w // C

    mesh = plsc.VectorSubcoreMesh(core_axis_name="c", subcore_axis_name="s")

    @functools.partial(
        pl.kernel,
        out_type=jax.ShapeDtypeStruct((rows, D_OUT), jnp.float32),
        mesh=mesh,
        scratch_types=[
            pltpu.VMEM((20, 8), jnp.float32),
            pltpu.VMEM((500, 16), jnp.float32),
            pltpu.VMEM((350, 16), jnp.float32),
            pltpu.VMEM((1024, 32), jnp.float32),
            pltpu.VMEM((2, C, 10), jnp.int32),
            pltpu.VMEM((2, C), jnp.int32),
            pltpu.VMEM((2, C, 32), jnp.float32),
            pltpu.VMEM((2, C, D_EMB), jnp.float32),
            pltpu.SemaphoreType.DMA,
            pltpu.SemaphoreType.DMA,
            pltpu.SemaphoreType.DMA,
            pltpu.SemaphoreType.DMA,
            pltpu.SemaphoreType.DMA,
            pltpu.SemaphoreType.DMA,
        ],
        compiler_params=pltpu.CompilerParams(
            use_tc_tiling_on_sc=False, needs_layout_passes=False),
    )
    def k(idx_pack, sp_flat, scal, sp_t, ty_t, it_t, ab_t, mv_t, out,
          ty_v, it_v, ab_v, mv_v, idxb, spi, spb, asmb,
          s_idx0, s_idx1, s_spg, s_wr0, s_wr1, s_scal):
        wid = lax.axis_index("s") * info.num_cores + lax.axis_index("c")
        w_base = wid * rows_per_w
        tabs = (ty_v, it_v, ab_v, mv_v)
        s_idx = (s_idx0, s_idx1)
        s_wr = (s_wr0, s_wr1)

        pltpu.sync_copy(ty_t, ty_v)
        pltpu.sync_copy(it_t, it_v)
        pltpu.sync_copy(ab_t, ab_v)
        pltpu.sync_copy(mv_t, mv_v)

        def issue_idx(ci, b):
            base = w_base + ci * C
            pltpu.async_copy(idx_pack.at[pl.ds(base, C)], idxb.at[b],
                             s_idx[b])
            pltpu.async_copy(sp_flat.at[pl.ds(base, C)], spi.at[b], s_idx[b])

        def wait_idx(b):
            pltpu.make_async_copy(idx_pack.at[pl.ds(0, C)], idxb.at[b],
                                  s_idx[b]).wait()
            pltpu.make_async_copy(sp_flat.at[pl.ds(0, C)], spi.at[b],
                                  s_idx[b]).wait()

        def issue_spg(b):
            pltpu.async_copy(sp_t.at[spi.at[b]], spb.at[b], s_spg)

        def wait_spg(b):
            pltpu.make_async_copy(sp_t.at[pl.ds(0, C)], spb.at[b],
                                  s_spg).wait()

        def wait_wr(b):
            pltpu.make_async_copy(asmb.at[b],
                                  out.at[pl.ds(w_base, C), pl.ds(0, D_EMB)],
                                  s_wr[b]).wait()

        def wait_scal():
            pltpu.make_async_copy(
                scal.at[pl.ds(w_base, C)],
                out.at[pl.ds(w_base, C), pl.ds(D_EMB, D_SCAL)],
                s_scal).wait()

        def assemble(b):
            asm_b = asmb.at[b]
            idx_b = idxb.at[b]
            spb_b = spb.at[b]

            def group(g, carry):
                rowv = lax.iota(jnp.int32, 16) + g * 16
                for col, off, width, t in _FIELDS:
                    csplat = jnp.full((16,), col, jnp.int32)
                    fidx = plsc.load_gather(idx_b, [rowv, csplat])
                    for c in range(width):
                        vals = plsc.load_gather(
                            tabs[t], [fidx, jnp.full((16,), c, jnp.int32)])
                        plsc.store_scatter(
                            asm_b, [rowv, jnp.full((16,), off + c, jnp.int32)],
                            vals)
                for j in range(16):
                    r = g * 16 + j
                    asm_b[r, pl.ds(0, 16)] = spb_b[r, pl.ds(0, 16)]
                    asm_b[r, pl.ds(16, 16)] = spb_b[r, pl.ds(16, 16)]
                return carry

            lax.fori_loop(0, G, group, 0)

        def body(ci, b):
            base = w_base + ci * C
            wait_spg(b)
            assemble(b)
            @pl.when(ci + 2 < n_chunks)
            def _():
                issue_idx(ci + 2, b)
            pltpu.async_copy(asmb.at[b],
                             out.at[pl.ds(base, C), pl.ds(0, D_EMB)], s_wr[b])
            pltpu.async_copy(scal.at[pl.ds(base, C)],
                             out.at[pl.ds(base, C), pl.ds(D_EMB, D_SCAL)],
                             s_scal)

            @pl.when(ci + 1 < n_chunks)
            def _():
                wait_idx(1 - b)
                issue_spg(1 - b)

            @pl.when(jnp.logical_and(ci + 1 < n_chunks, ci >= 1))
            def _():
                wait_wr(1 - b)
                wait_scal()

        issue_idx(0, 0)
        wait_idx(0)
        issue_spg(0)
        issue_idx(1, 1)

        def two_chunks(i, carry):
            body(2 * i, 0)
            body(2 * i + 1, 1)
            return carry

        lax.fori_loop(0, n_chunks // 2, two_chunks, 0)

        wait_wr(0)
        wait_wr(1)
        wait_scal()
        wait_scal()

    return k


def kernel(species_idx, type1_idx, type2_idx, tera_idx, item_idx, ability_idx,
           move_idx, scalars, species_embed, type_embed, item_embed,
           ability_embed, move_embed):
    b, n = species_idx.shape
    rows = b * n
    flat = lambda a: a.reshape(rows).astype(jnp.int32)
    mv = move_idx.reshape(rows, 4).astype(jnp.int32)
    sp = flat(species_idx)
    idx_pack = jnp.stack(
        [sp, flat(type1_idx), flat(type2_idx), flat(tera_idx),
         flat(item_idx), flat(ability_idx),
         mv[:, 0], mv[:, 1], mv[:, 2], mv[:, 3]], axis=1)
    k = _make_kernel(rows)
    out = k(idx_pack, sp, scalars.reshape(rows, D_SCAL), species_embed,
            type_embed, item_embed, ability_embed, move_embed)
    return out.reshape(b, n, D_OUT)


# speedup vs baseline: 2.5936x; 2.5936x over previous
import functools

import jax
import jax.numpy as jnp
from jax import lax
from jax.experimental import pallas as pl
from jax.experimental.pallas import tpu as pltpu
from jax.experimental.pallas import tpu_sc as plsc

D_OUT = 439
D_EMB = 216
D_SCAL = 223
C = 48
G = C // 16

_FIELDS = (
    (1, 32, 8, 0),
    (2, 40, 8, 0),
    (3, 48, 8, 0),
    (4, 56, 16, 1),
    (5, 72, 16, 2),
    (6, 88, 32, 3),
    (7, 120, 32, 3),
    (8, 152, 32, 3),
    (9, 184, 32, 3),
)


def _make_kernel(rows: int):
    info = plsc.get_sparse_core_info()
    nw = info.num_cores * info.num_subcores
    assert rows % (nw * 2 * C) == 0
    rows_per_w = rows // nw
    n_chunks = rows_per_w // C

    mesh = plsc.VectorSubcoreMesh(core_axis_name="c", subcore_axis_name="s")

    @functools.partial(
        pl.kernel,
        out_type=jax.ShapeDtypeStruct((rows, D_OUT), jnp.float32),
        mesh=mesh,
        scratch_types=[
            pltpu.VMEM((20, 8), jnp.float32),
            pltpu.VMEM((500, 16), jnp.float32),
            pltpu.VMEM((350, 16), jnp.float32),
            pltpu.VMEM((1024, 32), jnp.float32),
            pltpu.VMEM((2, C, 10), jnp.int32),
            pltpu.VMEM((2, C), jnp.int32),
            pltpu.VMEM((2, C, 32), jnp.float32),
            pltpu.VMEM((2, C, D_SCAL), jnp.float32),
            pltpu.VMEM((2, C, D_OUT), jnp.float32),
            pltpu.SemaphoreType.DMA,
            pltpu.SemaphoreType.DMA,
            pltpu.SemaphoreType.DMA,
            pltpu.SemaphoreType.DMA,
            pltpu.SemaphoreType.DMA,
            pltpu.SemaphoreType.DMA,
            pltpu.SemaphoreType.DMA,
            pltpu.SemaphoreType.DMA,
        ],
        compiler_params=pltpu.CompilerParams(
            use_tc_tiling_on_sc=False, needs_layout_passes=False),
    )
    def k(idx_pack, sp_flat, scal, sp_t, ty_t, it_t, ab_t, mv_t, out,
          ty_v, it_v, ab_v, mv_v, idxb, spi, spb, sclb, asmb,
          s_idx0, s_idx1, s_spg0, s_spg1, s_scl0, s_scl1, s_wr0, s_wr1):
        wid = lax.axis_index("s") * info.num_cores + lax.axis_index("c")
        w_base = wid * rows_per_w
        tabs = (ty_v, it_v, ab_v, mv_v)
        s_idx = (s_idx0, s_idx1)
        s_spg = (s_spg0, s_spg1)
        s_scl = (s_scl0, s_scl1)
        s_wr = (s_wr0, s_wr1)

        pltpu.sync_copy(ty_t, ty_v)
        pltpu.sync_copy(it_t, it_v)
        pltpu.sync_copy(ab_t, ab_v)
        pltpu.sync_copy(mv_t, mv_v)

        def issue_idx(ci, b):
            base = w_base + ci * C
            pltpu.async_copy(idx_pack.at[pl.ds(base, C)], idxb.at[b],
                             s_idx[b])
            pltpu.async_copy(sp_flat.at[pl.ds(base, C)], spi.at[b], s_idx[b])
            pltpu.async_copy(scal.at[pl.ds(base, C)], sclb.at[b], s_scl[b])

        def wait_idx(b):
            pltpu.make_async_copy(idx_pack.at[pl.ds(0, C)], idxb.at[b],
                                  s_idx[b]).wait()
            pltpu.make_async_copy(sp_flat.at[pl.ds(0, C)], spi.at[b],
                                  s_idx[b]).wait()

        def wait_scl(b):
            pltpu.make_async_copy(scal.at[pl.ds(0, C)], sclb.at[b],
                                  s_scl[b]).wait()

        def issue_spg(b):
            pltpu.async_copy(sp_t.at[spi.at[b]], spb.at[b], s_spg[b])

        def wait_spg(b):
            pltpu.make_async_copy(sp_t.at[pl.ds(0, C)], spb.at[b],
                                  s_spg[b]).wait()

        def wait_wr(b):
            pltpu.make_async_copy(asmb.at[b], out.at[pl.ds(w_base, C)],
                                  s_wr[b]).wait()

        def assemble(b):
            asm_b = asmb.at[b]
            idx_b = idxb.at[b]
            spb_b = spb.at[b]
            scl_b = sclb.at[b]

            def group(g, carry):
                rowv = lax.iota(jnp.int32, 16) + g * 16
                for col, off, width, t in _FIELDS:
                    csplat = jnp.full((16,), col, jnp.int32)
                    fidx = plsc.load_gather(idx_b, [rowv, csplat])
                    for c in range(width):
                        vals = plsc.load_gather(
                            tabs[t], [fidx, jnp.full((16,), c, jnp.int32)])
                        plsc.store_scatter(
                            asm_b, [rowv, jnp.full((16,), off + c, jnp.int32)],
                            vals)
                for j in range(16):
                    r = g * 16 + j
                    asm_b[r, pl.ds(0, 16)] = spb_b[r, pl.ds(0, 16)]
                    asm_b[r, pl.ds(16, 16)] = spb_b[r, pl.ds(16, 16)]
                    for t16 in range(13):
                        asm_b[r, pl.ds(D_EMB + 16 * t16, 16)] = (
                            scl_b[r, pl.ds(16 * t16, 16)])
                    asm_b[r, pl.ds(D_EMB + D_SCAL - 16, 16)] = (
                        scl_b[r, pl.ds(D_SCAL - 16, 16)])
                return carry

            lax.fori_loop(0, G, group, 0)

        def body(ci, b):
            base = w_base + ci * C
            wait_spg(b)
            wait_scl(b)
            @pl.when(ci >= 2)
            def _():
                wait_wr(b)
            assemble(b)
            @pl.when(ci + 2 < n_chunks)
            def _():
                issue_idx(ci + 2, b)
            pltpu.async_copy(asmb.at[b], out.at[pl.ds(base, C)], s_wr[b])

            @pl.when(ci + 1 < n_chunks)
            def _():
                wait_idx(1 - b)
                issue_spg(1 - b)

        issue_idx(0, 0)
        wait_idx(0)
        issue_spg(0)
        issue_idx(1, 1)

        def two_chunks(i, carry):
            body(2 * i, 0)
            body(2 * i + 1, 1)
            return carry

        lax.fori_loop(0, n_chunks // 2, two_chunks, 0)

        wait_wr(0)
        wait_wr(1)

    return k


def kernel(species_idx, type1_idx, type2_idx, tera_idx, item_idx, ability_idx,
           move_idx, scalars, species_embed, type_embed, item_embed,
           ability_embed, move_embed):
    b, n = species_idx.shape
    rows = b * n
    flat = lambda a: a.reshape(rows).astype(jnp.int32)
    mv = move_idx.reshape(rows, 4).astype(jnp.int32)
    sp = flat(species_idx)
    idx_pack = jnp.stack(
        [sp, flat(type1_idx), flat(type2_idx), flat(tera_idx),
         flat(item_idx), flat(ability_idx),
         mv[:, 0], mv[:, 1], mv[:, 2], mv[:, 3]], axis=1)
    k = _make_kernel(rows)
    out = k(idx_pack, sp, scalars.reshape(rows, D_SCAL), species_embed,
            type_embed, item_embed, ability_embed, move_embed)
    return out.reshape(b, n, D_OUT)


# speedup vs baseline: 3.2443x; 1.2509x over previous
import functools

import jax
import jax.numpy as jnp
from jax import lax
from jax.experimental import pallas as pl
from jax.experimental.pallas import tpu as pltpu
from jax.experimental.pallas import tpu_sc as plsc

D_OUT = 439
D_EMB = 216
D_SCAL = 223
N_IN_B = 12
CB = 4
C = CB * N_IN_B
G = C // 16

_FIELDS = (
    (1, 32, 8, 0),
    (2, 40, 8, 0),
    (3, 48, 8, 0),
    (4, 56, 16, 1),
    (5, 72, 16, 2),
    (6, 88, 32, 3),
    (7, 120, 32, 3),
    (8, 152, 32, 3),
    (9, 184, 32, 3),
)


def _make_kernel(n_batches: int):
    rows = n_batches * N_IN_B
    info = plsc.get_sparse_core_info()
    nw = info.num_cores * info.num_subcores
    assert n_batches % (nw * 2 * CB) == 0
    batches_per_w = n_batches // nw
    rows_per_w = batches_per_w * N_IN_B
    n_chunks = batches_per_w // CB

    mesh = plsc.VectorSubcoreMesh(core_axis_name="c", subcore_axis_name="s")

    @functools.partial(
        pl.kernel,
        out_type=jax.ShapeDtypeStruct((n_batches, N_IN_B, D_OUT),
                                      jnp.float32),
        mesh=mesh,
        scratch_types=[
            pltpu.VMEM((20, 8), jnp.float32),
            pltpu.VMEM((500, 16), jnp.float32),
            pltpu.VMEM((350, 16), jnp.float32),
            pltpu.VMEM((1024, 32), jnp.float32),
            pltpu.VMEM((2, C, 10), jnp.int32),
            pltpu.VMEM((2, C), jnp.int32),
            pltpu.VMEM((2, C, 32), jnp.float32),
            pltpu.VMEM((2, CB, N_IN_B, D_SCAL), jnp.float32),
            pltpu.VMEM((2, CB, N_IN_B, D_OUT), jnp.float32),
            pltpu.SemaphoreType.DMA,
            pltpu.SemaphoreType.DMA,
            pltpu.SemaphoreType.DMA,
            pltpu.SemaphoreType.DMA,
            pltpu.SemaphoreType.DMA,
            pltpu.SemaphoreType.DMA,
            pltpu.SemaphoreType.DMA,
            pltpu.SemaphoreType.DMA,
        ],
        compiler_params=pltpu.CompilerParams(
            use_tc_tiling_on_sc=False, needs_layout_passes=False),
    )
    def k(idx_pack, sp_flat, scal, sp_t, ty_t, it_t, ab_t, mv_t, out,
          ty_v, it_v, ab_v, mv_v, idxb, spi, spb, sclb, asmb,
          s_idx0, s_idx1, s_spg0, s_spg1, s_scl0, s_scl1, s_wr0, s_wr1):
        wid = lax.axis_index("s") * info.num_cores + lax.axis_index("c")
        w_base = wid * rows_per_w
        wb_base = wid * batches_per_w
        tabs = (ty_v, it_v, ab_v, mv_v)
        s_idx = (s_idx0, s_idx1)
        s_spg = (s_spg0, s_spg1)
        s_scl = (s_scl0, s_scl1)
        s_wr = (s_wr0, s_wr1)

        def qm_of(rowv):
            qv = jnp.zeros((16,), jnp.int32)
            for thresh in range(N_IN_B, C, N_IN_B):
                qv = qv + (rowv >= thresh).astype(jnp.int32)
            return qv, rowv - qv * N_IN_B

        pltpu.sync_copy(ty_t, ty_v)
        pltpu.sync_copy(it_t, it_v)
        pltpu.sync_copy(ab_t, ab_v)
        pltpu.sync_copy(mv_t, mv_v)

        def issue_idx(ci, b):
            base = w_base + ci * C
            bbase = wb_base + ci * CB
            pltpu.async_copy(idx_pack.at[pl.ds(base, C)], idxb.at[b],
                             s_idx[b])
            pltpu.async_copy(sp_flat.at[pl.ds(base, C)], spi.at[b], s_idx[b])
            pltpu.async_copy(scal.at[pl.ds(bbase, CB)], sclb.at[b], s_scl[b])

        def wait_idx(b):
            pltpu.make_async_copy(idx_pack.at[pl.ds(0, C)], idxb.at[b],
                                  s_idx[b]).wait()
            pltpu.make_async_copy(sp_flat.at[pl.ds(0, C)], spi.at[b],
                                  s_idx[b]).wait()

        def wait_scl(b):
            pltpu.make_async_copy(scal.at[pl.ds(0, CB)], sclb.at[b],
                                  s_scl[b]).wait()

        def issue_spg(b):
            pltpu.async_copy(sp_t.at[spi.at[b]], spb.at[b], s_spg[b])

        def wait_spg(b):
            pltpu.make_async_copy(sp_t.at[pl.ds(0, C)], spb.at[b],
                                  s_spg[b]).wait()

        def wait_wr(b):
            pltpu.make_async_copy(asmb.at[b], out.at[pl.ds(0, CB)],
                                  s_wr[b]).wait()

        def assemble(b):
            asm_b = asmb.at[b]
            idx_b = idxb.at[b]
            spb_b = spb.at[b]
            scl_b = sclb.at[b]

            for g in range(G):
                rowv = lax.iota(jnp.int32, 16) + g * 16
                qv, mv = qm_of(rowv)
                for col, off, width, t in _FIELDS:
                    csplat = jnp.full((16,), col, jnp.int32)
                    fidx = plsc.load_gather(idx_b, [rowv, csplat])
                    for c in range(width):
                        vals = plsc.load_gather(
                            tabs[t], [fidx, jnp.full((16,), c, jnp.int32)])
                        plsc.store_scatter(
                            asm_b,
                            [qv, mv, jnp.full((16,), off + c, jnp.int32)],
                            vals)

            for q in range(CB):
                for m in range(N_IN_B):
                    r = q * N_IN_B + m
                    asm_b[q, m, pl.ds(0, 16)] = spb_b[r, pl.ds(0, 16)]
                    asm_b[q, m, pl.ds(16, 16)] = spb_b[r, pl.ds(16, 16)]
                    for t16 in range(13):
                        asm_b[q, m, pl.ds(D_EMB + 16 * t16, 16)] = (
                            scl_b[q, m, pl.ds(16 * t16, 16)])
                    asm_b[q, m, pl.ds(D_EMB + D_SCAL - 16, 16)] = (
                        scl_b[q, m, pl.ds(D_SCAL - 16, 16)])

        def body(ci, b):
            bbase = wb_base + ci * CB
            wait_spg(b)
            wait_scl(b)
            @pl.when(ci >= 2)
            def _():
                wait_wr(b)
            assemble(b)
            @pl.when(ci + 2 < n_chunks)
            def _():
                issue_idx(ci + 2, b)
            pltpu.async_copy(asmb.at[b], out.at[pl.ds(bbase, CB)], s_wr[b])

            @pl.when(ci + 1 < n_chunks)
            def _():
                wait_idx(1 - b)
                issue_spg(1 - b)

        issue_idx(0, 0)
        wait_idx(0)
        issue_spg(0)
        issue_idx(1, 1)

        def two_chunks(i, carry):
            body(2 * i, 0)
            body(2 * i + 1, 1)
            return carry

        lax.fori_loop(0, n_chunks // 2, two_chunks, 0)

        wait_wr(0)
        wait_wr(1)

    return k


def kernel(species_idx, type1_idx, type2_idx, tera_idx, item_idx, ability_idx,
           move_idx, scalars, species_embed, type_embed, item_embed,
           ability_embed, move_embed):
    b, n = species_idx.shape
    rows = b * n
    flat = lambda a: a.reshape(rows).astype(jnp.int32)
    mv = move_idx.reshape(rows, 4).astype(jnp.int32)
    sp = flat(species_idx)
    idx_pack = jnp.stack(
        [sp, flat(type1_idx), flat(type2_idx), flat(tera_idx),
         flat(item_idx), flat(ability_idx),
         mv[:, 0], mv[:, 1], mv[:, 2], mv[:, 3]], axis=1)
    k = _make_kernel(b)
    return k(idx_pack, sp, scalars, species_embed, type_embed, item_embed,
             ability_embed, move_embed)


# speedup vs baseline: 3.3390x; 1.0292x over previous
import functools

import jax
import jax.numpy as jnp
from jax import lax
from jax.experimental import pallas as pl
from jax.experimental.pallas import tpu as pltpu
from jax.experimental.pallas import tpu_sc as plsc

D_OUT = 439
D_EMB = 216
D_SCAL = 223
N_IN_B = 12
CB = 4
C = CB * N_IN_B
G = C // 16

_FIELDS = (
    (0, 32, 8, 0),
    (1, 40, 8, 0),
    (2, 48, 8, 0),
    (3, 56, 16, 1),
    (4, 72, 16, 2),
)
_MOVE_OFF = 88
_MOVE_W = 32


def _make_kernel(n_batches: int):
    info = plsc.get_sparse_core_info()
    nw = info.num_cores * info.num_subcores
    assert n_batches % (nw * 2 * CB) == 0
    batches_per_w = n_batches // nw
    n_chunks = batches_per_w // CB

    mesh = plsc.VectorSubcoreMesh(core_axis_name="c", subcore_axis_name="s")

    @functools.partial(
        pl.kernel,
        out_type=jax.ShapeDtypeStruct((n_batches, N_IN_B, D_OUT),
                                      jnp.float32),
        mesh=mesh,
        scratch_types=[
            pltpu.VMEM((20, 8), jnp.float32),
            pltpu.VMEM((500, 16), jnp.float32),
            pltpu.VMEM((350, 16), jnp.float32),
            pltpu.VMEM((1024, 32), jnp.float32),
            pltpu.VMEM((2, 5, CB, N_IN_B), jnp.int32),
            pltpu.VMEM((2, CB, N_IN_B, 4), jnp.int32),
            pltpu.VMEM((2, CB, N_IN_B), jnp.int32),
            pltpu.VMEM((2, C), jnp.int32),
            pltpu.VMEM((2, C, 32), jnp.float32),
            pltpu.VMEM((2, CB, N_IN_B, D_SCAL), jnp.float32),
            pltpu.VMEM((2, CB, N_IN_B, D_OUT), jnp.float32),
            pltpu.SemaphoreType.DMA,
            pltpu.SemaphoreType.DMA,
            pltpu.SemaphoreType.DMA,
            pltpu.SemaphoreType.DMA,
            pltpu.SemaphoreType.DMA,
            pltpu.SemaphoreType.DMA,
            pltpu.SemaphoreType.DMA,
            pltpu.SemaphoreType.DMA,
        ],
        compiler_params=pltpu.CompilerParams(
            use_tc_tiling_on_sc=False, needs_layout_passes=False),
    )
    def k(sp_i, t1_i, t2_i, te_i, it_i, ab_i, mv_i, scal,
          sp_t, ty_t, it_t, ab_t, mv_t, out,
          ty_v, it_v, ab_v, mv_v, idxb, mvb, spl, spi, spb, sclb, asmb,
          s_idx0, s_idx1, s_spg0, s_spg1, s_scl0, s_scl1, s_wr0, s_wr1):
        wid = lax.axis_index("s") * info.num_cores + lax.axis_index("c")
        wb_base = wid * batches_per_w
        tabs = (ty_v, it_v, ab_v, mv_v)
        fields = (t1_i, t2_i, te_i, it_i, ab_i)
        s_idx = (s_idx0, s_idx1)
        s_spg = (s_spg0, s_spg1)
        s_scl = (s_scl0, s_scl1)
        s_wr = (s_wr0, s_wr1)

        def qm_of(rowv):
            qv = jnp.zeros((16,), jnp.int32)
            for thresh in range(N_IN_B, C, N_IN_B):
                qv = qv + (rowv >= thresh).astype(jnp.int32)
            return qv, rowv - qv * N_IN_B

        pltpu.sync_copy(ty_t, ty_v)
        pltpu.sync_copy(it_t, it_v)
        pltpu.sync_copy(ab_t, ab_v)
        pltpu.sync_copy(mv_t, mv_v)

        def issue_idx(ci, b):
            bbase = wb_base + ci * CB
            sl = pl.ds(bbase, CB)
            for fs, f_ref in enumerate(fields):
                pltpu.async_copy(f_ref.at[sl], idxb.at[b, fs], s_idx[b])
            pltpu.async_copy(mv_i.at[sl], mvb.at[b], s_idx[b])
            pltpu.async_copy(sp_i.at[sl], spl.at[b], s_idx[b])
            pltpu.async_copy(scal.at[sl], sclb.at[b], s_scl[b])

        def wait_idx(b):
            for fs, f_ref in enumerate(fields):
                pltpu.make_async_copy(f_ref.at[pl.ds(0, CB)], idxb.at[b, fs],
                                      s_idx[b]).wait()
            pltpu.make_async_copy(mv_i.at[pl.ds(0, CB)], mvb.at[b],
                                  s_idx[b]).wait()
            pltpu.make_async_copy(sp_i.at[pl.ds(0, CB)], spl.at[b],
                                  s_idx[b]).wait()

        def wait_scl(b):
            pltpu.make_async_copy(scal.at[pl.ds(0, CB)], sclb.at[b],
                                  s_scl[b]).wait()

        def compact_species(b):
            spl_b = spl.at[b]
            spi_b = spi.at[b]
            for g in range(G):
                rowv = lax.iota(jnp.int32, 16) + g * 16
                qv, mv = qm_of(rowv)
                vals = plsc.load_gather(spl_b, [qv, mv])
                spi_b[pl.ds(g * 16, 16)] = vals

        def issue_spg(b):
            pltpu.async_copy(sp_t.at[spi.at[b]], spb.at[b], s_spg[b])

        def wait_spg(b):
            pltpu.make_async_copy(sp_t.at[pl.ds(0, C)], spb.at[b],
                                  s_spg[b]).wait()

        def wait_wr(b):
            pltpu.make_async_copy(asmb.at[b], out.at[pl.ds(0, CB)],
                                  s_wr[b]).wait()

        def assemble(b):
            asm_b = asmb.at[b]
            spb_b = spb.at[b]
            scl_b = sclb.at[b]

            for g in range(G):
                rowv = lax.iota(jnp.int32, 16) + g * 16
                qv, mv = qm_of(rowv)
                for fs, off, width, t in _FIELDS:
                    fidx = plsc.load_gather(idxb.at[b, fs], [qv, mv])
                    for c in range(width):
                        vals = plsc.load_gather(
                            tabs[t], [fidx, jnp.full((16,), c, jnp.int32)])
                        plsc.store_scatter(
                            asm_b,
                            [qv, mv, jnp.full((16,), off + c, jnp.int32)],
                            vals)
                for k4 in range(4):
                    fidx = plsc.load_gather(
                        mvb.at[b], [qv, mv, jnp.full((16,), k4, jnp.int32)])
                    off = _MOVE_OFF + k4 * _MOVE_W
                    for c in range(_MOVE_W):
                        vals = plsc.load_gather(
                            mv_v, [fidx, jnp.full((16,), c, jnp.int32)])
                        plsc.store_scatter(
                            asm_b,
                            [qv, mv, jnp.full((16,), off + c, jnp.int32)],
                            vals)

            for q in range(CB):
                for m in range(N_IN_B):
                    r = q * N_IN_B + m
                    asm_b[q, m, pl.ds(0, 16)] = spb_b[r, pl.ds(0, 16)]
                    asm_b[q, m, pl.ds(16, 16)] = spb_b[r, pl.ds(16, 16)]
                    for t16 in range(13):
                        asm_b[q, m, pl.ds(D_EMB + 16 * t16, 16)] = (
                            scl_b[q, m, pl.ds(16 * t16, 16)])
                    asm_b[q, m, pl.ds(D_EMB + D_SCAL - 16, 16)] = (
                        scl_b[q, m, pl.ds(D_SCAL - 16, 16)])

        def body(ci, b):
            bbase = wb_base + ci * CB
            wait_spg(b)
            wait_scl(b)
            @pl.when(ci >= 2)
            def _():
                wait_wr(b)
            assemble(b)
            @pl.when(ci + 2 < n_chunks)
            def _():
                issue_idx(ci + 2, b)
            pltpu.async_copy(asmb.at[b], out.at[pl.ds(bbase, CB)], s_wr[b])

            @pl.when(ci + 1 < n_chunks)
            def _():
                wait_idx(1 - b)
                compact_species(1 - b)
                issue_spg(1 - b)

        issue_idx(0, 0)
        wait_idx(0)
        compact_species(0)
        issue_spg(0)
        issue_idx(1, 1)

        def two_chunks(i, carry):
            body(2 * i, 0)
            body(2 * i + 1, 1)
            return carry

        lax.fori_loop(0, n_chunks // 2, two_chunks, 0)

        wait_wr(0)
        wait_wr(1)

    return k


def kernel(species_idx, type1_idx, type2_idx, tera_idx, item_idx, ability_idx,
           move_idx, scalars, species_embed, type_embed, item_embed,
           ability_embed, move_embed):
    b, n = species_idx.shape
    i32 = lambda a: a.astype(jnp.int32)
    k = _make_kernel(b)
    return k(i32(species_idx), i32(type1_idx), i32(type2_idx), i32(tera_idx),
             i32(item_idx), i32(ability_idx), i32(move_idx), scalars,
             species_embed, type_embed, item_embed, ability_embed, move_embed)
